# baseline (speedup 1.0000x reference)
import functools

import jax
import jax.numpy as jnp
from jax import lax
from jax.experimental import pallas as pl
from jax.experimental.pallas import tpu as pltpu
from jax.experimental.pallas import tpu_sc as plsc

NW = 32
L = 16


@functools.lru_cache(maxsize=None)
def _build(B, S, D, V):
    SPW = B // NW
    C = 8
    NCHUNK = SPW // C
    CHW = C * S
    GB = 64
    NBLK = CHW // GB

    mesh = plsc.VectorSubcoreMesh(core_axis_name="c", subcore_axis_name="s")

    @functools.partial(
        pl.kernel,
        out_type=jax.ShapeDtypeStruct((B * D,), jnp.float32),
        mesh=mesh,
        scratch_types=[
            pltpu.VMEM((CHW,), jnp.int32),
            pltpu.VMEM((CHW,), jnp.int32),
            pltpu.VMEM((NBLK, GB), jnp.int32),
            pltpu.VMEM((CHW, D), jnp.float32),
            pltpu.VMEM((C * D,), jnp.float32),
            pltpu.VMEM((D,), jnp.float32),
            pltpu.SemaphoreType.DMA,
        ],
        compiler_params=pltpu.CompilerParams(
            use_tc_tiling_on_sc=False, needs_layout_passes=False),
    )
    def enc(tok_hbm, mask_hbm, w0_hbm, table_hbm, out_hbm,
            tok_v, mask_v, idx_v, rows_v, outb_v, w0_v, sem):
        cid = lax.axis_index("c")
        sid = lax.axis_index("s")
        wid = sid * 2 + cid

        pltpu.sync_copy(w0_hbm, w0_v)
        w0a = w0_v[pl.ds(0, L)]
        w0b = w0_v[pl.ds(L, L)]
        lanes = lax.iota(jnp.int32, L)
        tailm = jnp.minimum(jnp.maximum(lanes - (L - S % L - 1), 0), 1)

        def chunk_body(ci, carry):
            tbase = (wid * SPW + ci * C) * S
            pltpu.sync_copy(tok_hbm.at[pl.ds(tbase, CHW)], tok_v)
            pltpu.sync_copy(mask_hbm.at[pl.ds(tbase, CHW)], mask_v)

            def idx_body(bb, c2):
                for u in range(GB // L):
                    off = bb * GB + u * L
                    t = tok_v[pl.ds(off, L)]
                    m = mask_v[pl.ds(off, L)]
                    idx_v[bb, pl.ds(u * L, L)] = t * m
                return c2
            lax.fori_loop(0, NBLK, idx_body, 0)

            def fire(bb, c2):
                pltpu.make_async_copy(
                    table_hbm.at[idx_v.at[bb]],
                    rows_v.at[pl.ds(bb * GB, GB)], sem).start()
                return c2
            lax.fori_loop(0, NBLK, fire, 0)

            def drain(bb, c2):
                pltpu.make_async_copy(
                    table_hbm.at[idx_v.at[0]],
                    rows_v.at[pl.ds(0, GB)], sem).wait()
                return c2
            lax.fori_loop(0, NBLK, drain, 0)

            def samp_body(s, c2):
                mb = s * S
                cnt_vec = mask_v[pl.ds(mb, L)]
                for q in range(1, S // L):
                    cnt_vec = cnt_vec + mask_v[pl.ds(mb + q * L, L)]
                cnt_vec = cnt_vec + mask_v[pl.ds(mb + S - L, L)] * tailm
                cnt = jnp.sum(cnt_vec)

                acc0 = rows_v[mb, pl.ds(0, L)]
                acc1 = rows_v[mb, pl.ds(L, L)]

                cntf = jnp.full((L,), cnt.astype(jnp.float32))
                n0 = jnp.float32(S) - cntf
                scale = jnp.float32(1.0) / jnp.maximum(cntf, 1.0)
                outb_v[pl.ds(s * D, L)] = (acc0 - n0 * w0a) * scale
                outb_v[pl.ds(s * D + L, L)] = (acc1 - n0 * w0b) * scale
                return c2
            lax.fori_loop(0, C, samp_body, 0)

            pltpu.sync_copy(
                outb_v, out_hbm.at[pl.ds((wid * SPW + ci * C) * D, C * D)])
            return carry
        lax.fori_loop(0, NCHUNK, chunk_body, 0)

    return enc


def kernel(tokens, mask, W):
    B, S = tokens.shape
    V, D = W.shape
    enc = _build(B, S, D, V)
    out = enc(tokens.reshape(-1),
              mask.astype(jnp.int32).reshape(-1),
              W[0],
              W)
    return out.reshape(B, D)


# speedup vs baseline: 10.8062x; 10.8062x over previous
import functools

import jax
import jax.numpy as jnp
from jax import lax
from jax.experimental import pallas as pl
from jax.experimental.pallas import tpu as pltpu
from jax.experimental.pallas import tpu_sc as plsc

NW = 32
L = 16


@functools.lru_cache(maxsize=None)
def _build(B, S, D, V):
    SPW = B // NW
    C = 8
    NCHUNK = SPW // C
    CHW = C * S
    GB = 80
    NBLK = CHW // GB
    NZ = 256

    mesh = plsc.VectorSubcoreMesh(core_axis_name="c", subcore_axis_name="s")

    @functools.partial(
        pl.kernel,
        out_type=jax.ShapeDtypeStruct((B * D,), jnp.float32),
        mesh=mesh,
        scratch_types=[
            pltpu.VMEM((CHW,), jnp.int32),
            pltpu.VMEM((CHW,), jnp.int32),
            pltpu.VMEM((NBLK, GB), jnp.int32),
            pltpu.VMEM((CHW, D), jnp.float32),
            pltpu.VMEM((C * D,), jnp.float32),
            pltpu.SemaphoreType.DMA,
        ],
        compiler_params=pltpu.CompilerParams(
            use_tc_tiling_on_sc=False, needs_layout_passes=False),
    )
    def enc(tok_hbm, mask_hbm, table_hbm, out_hbm,
            tok_v, mask_v, idx_v, rows_v, outb_v, sem):
        cid = lax.axis_index("c")
        sid = lax.axis_index("s")
        wid = sid * 2 + cid

        lanes = lax.iota(jnp.int32, L)
        tailm = jnp.minimum(jnp.maximum(lanes - (L - S % L - 1), 0), 1)

        def chunk_body(ci, carry):
            tbase = (wid * SPW + ci * C) * S
            pltpu.sync_copy(tok_hbm.at[pl.ds(tbase, CHW)], tok_v)
            pltpu.sync_copy(mask_hbm.at[pl.ds(tbase, CHW)], mask_v)

            def idx_body(bb, c2):
                for u in range(GB // L):
                    off = bb * GB + u * L
                    t = tok_v[pl.ds(off, L)]
                    m = mask_v[pl.ds(off, L)]
                    dummy = V + ((off % NZ + lanes) % NZ)
                    idx_v[bb, pl.ds(u * L, L)] = t * m + dummy * (1 - m)
                return c2
            lax.fori_loop(0, NBLK, idx_body, 0)

            def fire(bb, c2):
                pltpu.make_async_copy(
                    table_hbm.at[idx_v.at[bb]],
                    rows_v.at[pl.ds(bb * GB, GB)], sem).start()
                return c2
            lax.fori_loop(0, NBLK, fire, 0)

            def drain(bb, c2):
                pltpu.make_async_copy(
                    table_hbm.at[idx_v.at[0]],
                    rows_v.at[pl.ds(0, GB)], sem).wait()
                return c2
            lax.fori_loop(0, NBLK, drain, 0)

            def samp_body(s, c2):
                mb = s * S
                cnt_vec = mask_v[pl.ds(mb, L)]
                for q in range(1, S // L):
                    cnt_vec = cnt_vec + mask_v[pl.ds(mb + q * L, L)]
                cnt_vec = cnt_vec + mask_v[pl.ds(mb + S - L, L)] * tailm
                cnt = jnp.sum(cnt_vec)

                acc0 = jnp.zeros((L,), jnp.float32)
                acc1 = jnp.zeros((L,), jnp.float32)
                for j in range(S):
                    acc0 = acc0 + rows_v[mb + j, pl.ds(0, L)]
                    acc1 = acc1 + rows_v[mb + j, pl.ds(L, L)]

                cntf = jnp.full((L,), cnt.astype(jnp.float32))
                scale = jnp.float32(1.0) / jnp.maximum(cntf, 1.0)
                outb_v[pl.ds(s * D, L)] = acc0 * scale
                outb_v[pl.ds(s * D + L, L)] = acc1 * scale
                return c2
            lax.fori_loop(0, C, samp_body, 0)

            pltpu.sync_copy(
                outb_v, out_hbm.at[pl.ds((wid * SPW + ci * C) * D, C * D)])
            return carry
        lax.fori_loop(0, NCHUNK, chunk_body, 0)

    return enc


def kernel(tokens, mask, W):
    B, S = tokens.shape
    V, D = W.shape
    enc = _build(B, S, D, V)
    W_ext = jnp.concatenate([W, jnp.zeros((256, D), jnp.float32)], axis=0)
    out = enc(tokens.reshape(-1),
              mask.astype(jnp.int32).reshape(-1),
              W_ext)
    return out.reshape(B, D)


# speedup vs baseline: 17.5513x; 1.6242x over previous
import functools

import jax
import jax.numpy as jnp
from jax import lax
from jax.experimental import pallas as pl
from jax.experimental.pallas import tpu as pltpu
from jax.experimental.pallas import tpu_sc as plsc

NW = 32
L = 16


@functools.lru_cache(maxsize=None)
def _build(B, S, D, V):
    SPW = B // NW
    C = 8
    NCHUNK = SPW // C
    CHW = C * S
    GB = 80
    NBLK = CHW // GB

    mesh = plsc.VectorSubcoreMesh(core_axis_name="c", subcore_axis_name="s")

    @functools.partial(
        pl.kernel,
        out_type=jax.ShapeDtypeStruct((B * D,), jnp.float32),
        mesh=mesh,
        scratch_types=[
            pltpu.VMEM((CHW,), jnp.int32),
            pltpu.VMEM((CHW,), jnp.int32),
            pltpu.VMEM((CHW,), jnp.int32),
            pltpu.VMEM((CHW, D), jnp.float32),
            pltpu.VMEM((C * D,), jnp.float32),
            pltpu.SemaphoreType.DMA,
        ],
        compiler_params=pltpu.CompilerParams(
            use_tc_tiling_on_sc=False, needs_layout_passes=False),
    )
    def enc(tok_hbm, mask_hbm, table_hbm, out_hbm,
            tok_v, mask_v, idx_v, rows_v, outb_v, sem):
        cid = lax.axis_index("c")
        sid = lax.axis_index("s")
        wid = sid * 2 + cid

        lanes = lax.iota(jnp.int32, L)
        first8 = jnp.minimum(jnp.maximum(8 - lanes, 0), 1)
        last8 = 1 - first8
        zero16 = jnp.zeros((L,), jnp.int32)
        zf16 = jnp.zeros((L,), jnp.float32)

        def seed_body(kk, c2):
            for u in range(4):
                off = (kk * 4 + u) * L
                idx_v[pl.ds(off, L)] = lanes + off
            return c2
        lax.fori_loop(0, CHW // (4 * L), seed_body, 0)

        def chunk_body(ci, carry):
            tbase = (wid * SPW + ci * C) * S
            pltpu.sync_copy(tok_hbm.at[pl.ds(tbase, CHW)], tok_v)
            pltpu.sync_copy(mask_hbm.at[pl.ds(tbase, CHW)], mask_v)

            def emit(off_vec, k, sub):
                t = tok_v[pl.ds(k * L, L)]
                m = mask_v[pl.ds(k * L, L)]
                if sub == 0:
                    ms = m * first8
                elif sub == 1:
                    ms = m * last8
                else:
                    ms = m
                mb = ms != zero16
                pos = off_vec + plsc.cumsum(ms) - 1
                plsc.store_scatter(idx_v, [pos], t, mask=mb)
                return off_vec + plsc.all_reduce_population_count(mb)

            off_vec = zero16
            ends = []
            vpp = (2 * S) // L
            for p in range(C // 2):
                base = p * vpp
                for k in range(S // L):
                    off_vec = emit(off_vec, base + k, 2)
                off_vec = emit(off_vec, base + S // L, 0)
                ends.append(jnp.max(off_vec))
                off_vec = emit(off_vec, base + S // L, 1)
                for k in range(S // L + 1, vpp):
                    off_vec = emit(off_vec, base + k, 2)
                ends.append(jnp.max(off_vec))
            ntot = ends[-1]

            def fire(bb, c2):
                @pl.when(bb * GB < ntot)
                def _():
                    pltpu.make_async_copy(
                        table_hbm.at[idx_v.at[pl.ds(bb * GB, GB)]],
                        rows_v.at[pl.ds(bb * GB, GB)], sem).start()
                return c2
            lax.fori_loop(0, NBLK, fire, 0)

            def drain(bb, c2):
                @pl.when(bb * GB < ntot)
                def _():
                    pltpu.make_async_copy(
                        table_hbm.at[idx_v.at[pl.ds(0, GB)]],
                        rows_v.at[pl.ds(0, GB)], sem).wait()
                return c2
            lax.fori_loop(0, NBLK, drain, 0)

            for s in range(C):
                off_s = jnp.int32(0) if s == 0 else ends[s - 1]
                cnt_s = ends[s] - off_s

                def g8(g, a, off_s=off_s):
                    a0, a1 = a
                    rb = off_s + g * 8
                    for r in range(8):
                        a0 = a0 + rows_v[rb + r, pl.ds(0, L)]
                        a1 = a1 + rows_v[rb + r, pl.ds(L, L)]
                    return (a0, a1)
                acc0, acc1 = lax.fori_loop(0, cnt_s // 8, g8, (zf16, zf16))

                def g1(j, a, off_s=off_s):
                    a0, a1 = a
                    return (a0 + rows_v[off_s + j, pl.ds(0, L)],
                            a1 + rows_v[off_s + j, pl.ds(L, L)])
                acc0, acc1 = lax.fori_loop(
                    (cnt_s // 8) * 8, cnt_s, g1, (acc0, acc1))

                cntf = jnp.full((L,), cnt_s.astype(jnp.float32))
                scale = jnp.float32(1.0) / jnp.maximum(cntf, 1.0)
                outb_v[pl.ds(s * D, L)] = acc0 * scale
                outb_v[pl.ds(s * D + L, L)] = acc1 * scale

            pltpu.sync_copy(
                outb_v, out_hbm.at[pl.ds((wid * SPW + ci * C) * D, C * D)])
            return carry
        lax.fori_loop(0, NCHUNK, chunk_body, 0)

    return enc


def kernel(tokens, mask, W):
    B, S = tokens.shape
    V, D = W.shape
    enc = _build(B, S, D, V)
    out = enc(tokens.reshape(-1),
              mask.astype(jnp.int32).reshape(-1),
              W)
    return out.reshape(B, D)


# speedup vs baseline: 20.4263x; 1.1638x over previous
import functools

import jax
import jax.numpy as jnp
from jax import lax
from jax.experimental import pallas as pl
from jax.experimental.pallas import tpu as pltpu
from jax.experimental.pallas import tpu_sc as plsc

NW = 32
L = 16


@functools.lru_cache(maxsize=None)
def _build(B, S, D, V):
    SPW = B // NW
    C = 8
    NCHUNK = SPW // C
    CHW = C * S
    GB = 80
    NBLK = CHW // GB

    mesh = plsc.VectorSubcoreMesh(core_axis_name="c", subcore_axis_name="s")

    @functools.partial(
        pl.kernel,
        out_type=jax.ShapeDtypeStruct((B * D,), jnp.float32),
        mesh=mesh,
        scratch_types=[
            pltpu.VMEM((2 * CHW,), jnp.int32),
            pltpu.VMEM((2 * CHW,), jnp.int32),
            pltpu.VMEM((2 * CHW,), jnp.int32),
            pltpu.VMEM((2 * CHW, D), jnp.float32),
            pltpu.VMEM((C * D,), jnp.float32),
            pltpu.SemaphoreType.DMA,
            pltpu.SemaphoreType.DMA,
            pltpu.SemaphoreType.DMA,
            pltpu.SemaphoreType.DMA,
        ],
        compiler_params=pltpu.CompilerParams(
            use_tc_tiling_on_sc=False, needs_layout_passes=False),
    )
    def enc(tok_hbm, mask_hbm, table_hbm, out_hbm,
            tok_v, mask_v, idx_v, rows_v, outb_v,
            sem_r0, sem_r1, sem_t0, sem_t1):
        cid = lax.axis_index("c")
        sid = lax.axis_index("s")
        wid = sid * 2 + cid

        lanes = lax.iota(jnp.int32, L)
        first8 = jnp.minimum(jnp.maximum(8 - lanes, 0), 1)
        last8 = 1 - first8
        zero16 = jnp.zeros((L,), jnp.int32)
        zf16 = jnp.zeros((L,), jnp.float32)
        sem_r = (sem_r0, sem_r1)
        sem_t = (sem_t0, sem_t1)

        def seed_body(kk, c2):
            for u in range(4):
                off = (kk * 4 + u) * L
                idx_v[pl.ds(off, L)] = lanes + off
            return c2
        lax.fori_loop(0, 2 * CHW // (4 * L), seed_body, 0)

        def load_tm(ci, b):
            tbase = (wid * SPW + ci * C) * S
            tb = b * CHW
            pltpu.make_async_copy(
                tok_hbm.at[pl.ds(tbase, CHW)],
                tok_v.at[pl.ds(tb, CHW)], sem_t[b]).start()
            pltpu.make_async_copy(
                mask_hbm.at[pl.ds(tbase, CHW)],
                mask_v.at[pl.ds(tb, CHW)], sem_t[b]).start()

        def wait_tm(b):
            tb = b * CHW
            pltpu.make_async_copy(
                tok_hbm.at[pl.ds(0, CHW)],
                tok_v.at[pl.ds(tb, CHW)], sem_t[b]).wait()
            pltpu.make_async_copy(
                mask_hbm.at[pl.ds(0, CHW)],
                mask_v.at[pl.ds(tb, CHW)], sem_t[b]).wait()

        def compact(b):
            tb = b * CHW

            def emit(off_vec, k, sub):
                t = tok_v[pl.ds(tb + k * L, L)]
                m = mask_v[pl.ds(tb + k * L, L)]
                if sub == 0:
                    ms = m * first8
                elif sub == 1:
                    ms = m * last8
                else:
                    ms = m
                mb = ms != zero16
                pos = off_vec + plsc.cumsum(ms) - 1
                plsc.store_scatter(idx_v.at[pl.ds(tb, CHW)], [pos], t,
                                   mask=mb)
                return off_vec + plsc.all_reduce_population_count(mb)

            off_vec = zero16
            ends = []
            vpp = (2 * S) // L
            for p in range(C // 2):
                base = p * vpp
                for k in range(S // L):
                    off_vec = emit(off_vec, base + k, 2)
                off_vec = emit(off_vec, base + S // L, 0)
                ends.append(jnp.max(off_vec))
                off_vec = emit(off_vec, base + S // L, 1)
                for k in range(S // L + 1, vpp):
                    off_vec = emit(off_vec, base + k, 2)
                ends.append(jnp.max(off_vec))
            return tuple(ends)

        def fire_rows(b, ntot):
            tb = b * CHW

            def fire(bb, c2):
                @pl.when(bb * GB < ntot)
                def _():
                    pltpu.make_async_copy(
                        table_hbm.at[idx_v.at[pl.ds(tb + bb * GB, GB)]],
                        rows_v.at[pl.ds(tb + bb * GB, GB)], sem_r[b]).start()
                return c2
            lax.fori_loop(0, NBLK, fire, 0)

        def drain_rows(b, ntot):
            tb = b * CHW

            def drain(bb, c2):
                @pl.when(bb * GB < ntot)
                def _():
                    pltpu.make_async_copy(
                        table_hbm.at[idx_v.at[pl.ds(tb, GB)]],
                        rows_v.at[pl.ds(tb, GB)], sem_r[b]).wait()
                return c2
            lax.fori_loop(0, NBLK, drain, 0)

        def accum(b, ends, ci):
            rbb = b * CHW
            for s in range(C):
                off_s = jnp.int32(0) if s == 0 else ends[s - 1]
                cnt_s = ends[s] - off_s
                rb0 = rbb + off_s

                def g8(g, a, rb0=rb0):
                    a0, a1 = a
                    rb = rb0 + g * 8
                    for r in range(8):
                        a0 = a0 + rows_v[rb + r, pl.ds(0, L)]
                        a1 = a1 + rows_v[rb + r, pl.ds(L, L)]
                    return (a0, a1)
                acc0, acc1 = lax.fori_loop(0, cnt_s // 8, g8, (zf16, zf16))

                def g1(j, a, rb0=rb0):
                    a0, a1 = a
                    return (a0 + rows_v[rb0 + j, pl.ds(0, L)],
                            a1 + rows_v[rb0 + j, pl.ds(L, L)])
                acc0, acc1 = lax.fori_loop(
                    (cnt_s // 8) * 8, cnt_s, g1, (acc0, acc1))

                cntf = jnp.full((L,), cnt_s.astype(jnp.float32))
                scale = jnp.float32(1.0) / jnp.maximum(cntf, 1.0)
                outb_v[pl.ds(s * D, L)] = acc0 * scale
                outb_v[pl.ds(s * D + L, L)] = acc1 * scale
            pltpu.sync_copy(
                outb_v, out_hbm.at[pl.ds((wid * SPW + ci * C) * D, C * D)])

        load_tm(0, 0)
        wait_tm(0)
        ends0 = compact(0)
        fire_rows(0, ends0[-1])
        load_tm(1, 1)

        def pair_body(pi, carry):
            ci = 2 * pi
            wait_tm(1)
            ends_b1 = compact(1)
            fire_rows(1, ends_b1[-1])
            load_tm(ci + 2, 0)
            drain_rows(0, carry[-1])
            accum(0, carry, ci)

            wait_tm(0)
            ends_b0 = compact(0)
            fire_rows(0, ends_b0[-1])
            load_tm(ci + 3, 1)
            drain_rows(1, ends_b1[-1])
            accum(1, ends_b1, ci + 1)
            return ends_b0
        carry = lax.fori_loop(0, NCHUNK // 2 - 1, pair_body, ends0)

        wait_tm(1)
        ends_last = compact(1)
        fire_rows(1, ends_last[-1])
        drain_rows(0, carry[-1])
        accum(0, carry, NCHUNK - 2)
        drain_rows(1, ends_last[-1])
        accum(1, ends_last, NCHUNK - 1)

    return enc


def kernel(tokens, mask, W):
    B, S = tokens.shape
    V, D = W.shape
    enc = _build(B, S, D, V)
    out = enc(tokens.reshape(-1),
              mask.astype(jnp.int32).reshape(-1),
              W)
    return out.reshape(B, D)
